# baseline (speedup 1.0000x reference)
import functools

import jax
import jax.numpy as jnp
from jax import lax
from jax.experimental import pallas as pl
from jax.experimental.pallas import tpu as pltpu
from jax.experimental.pallas import tpu_sc as plsc

_B = 16384
_D = 64
_P = 8192
_PR = 4096
_L1_REG = 0.001
_ALPHA = 0.2

_NC = 2
_NS = 16
_NW = _NC * _NS
_CHUNK = 128


def _rsqrt_nr(x):
    xi = lax.bitcast_convert_type(x, jnp.int32)
    yi = jnp.int32(0x5F3759DF) - lax.shift_right_logical(xi, 1)
    y = lax.bitcast_convert_type(yi, jnp.float32)
    for _ in range(3):
        y = y * (1.5 - 0.5 * x * y * y)
    return y


def _sc_pair_kernel(s_hbm, ns_hbm, ps_hbm, nps_hbm, dis_hbm, sa_hbm, ref_hbm,
                    out_hbm, prs, ia, ib, ga, gb, gc, gd, outb, sem):
    cid = lax.axis_index("c")
    sid = lax.axis_index("s")
    wid = sid * _NC + cid

    iota16 = lax.iota(jnp.int32, 16)
    zz = jnp.zeros((16,), jnp.int32)
    oo = jnp.full((16,), 1, jnp.int32)
    zf = jnp.zeros((16,), jnp.float32)

    def load_pair_cols(pairs_hbm, base):
        pltpu.sync_copy(pairs_hbm.at[pl.ds(2 * base, 2 * _CHUNK)], prs)
        for j in range(_CHUNK // 16):
            r = (iota16 + j * 16) * 2
            ia[pl.ds(j * 16, 16)] = plsc.load_gather(prs, [r])
            ib[pl.ds(j * 16, 16)] = plsc.load_gather(prs, [r + 1])

    def sqdist_16(xa, xb, j):
        row = iota16 + j * 16

        def dc_body(dc, acc):
            a = acc
            for dd in range(16):
                col = dc * 16 + dd
                cv = jnp.full((16,), col, jnp.int32)
                va = plsc.load_gather(xa, [row, cv])
                vb = plsc.load_gather(xb, [row, cv])
                t = va - vb
                a = a + t * t
            return a

        return lax.fori_loop(0, _D // 16, dc_body, zf)

    def sa_quads_16(j):
        row = iota16 + j * 16

        def dc_body(dc, carry):
            a_s, a_df, a_qa, a_qb = carry
            for dd in range(16):
                col = dc * 16 + dd
                cv = jnp.full((16,), col, jnp.int32)
                sa = plsc.load_gather(ga, [row, cv])
                sb = plsc.load_gather(gb, [row, cv])
                na = plsc.load_gather(gc, [row, cv])
                nb = plsc.load_gather(gd, [row, cv])
                dsv = sa - sb
                a_s = a_s + dsv * dsv
                da = na - sa
                db = nb - sb
                dd_ = da - db
                a_df = a_df + dd_ * dd_
                a_qa = a_qa + da * da
                a_qb = a_qb + db * db
            return (a_s, a_df, a_qa, a_qb)

        return lax.fori_loop(0, _D // 16, dc_body, (zf, zf, zf, zf))

    caus_acc = zf
    prop_acc = zf
    rep_acc = zf
    ref_acc = zf

    for (st, nx) in ((s_hbm, ns_hbm), (ps_hbm, nps_hbm)):
        def sa_chunk(c, carry):
            rep_c, prop_c = carry
            base = wid * (_P // _NW) + c * _CHUNK
            load_pair_cols(sa_hbm, base)
            h1 = pltpu.async_copy(st.at[ia], ga, sem)
            h2 = pltpu.async_copy(st.at[ib], gb, sem)
            h3 = pltpu.async_copy(nx.at[ia], gc, sem)
            h4 = pltpu.async_copy(nx.at[ib], gd, sem)
            h1.wait(); h2.wait(); h3.wait(); h4.wait()

            def jbody(j, carry2):
                rep_j, prop_j = carry2
                acc_s, acc_df, acc_qa, acc_qb = sa_quads_16(j)
                sim = jnp.exp(-acc_s)
                rep_j = rep_j + sim * acc_df
                norm_a = acc_qa * _rsqrt_nr(acc_qa)
                norm_b = acc_qb * _rsqrt_nr(acc_qb)
                dn = norm_a - norm_b
                prop_j = prop_j + dn * dn
                return (rep_j, prop_j)

            return lax.fori_loop(0, _CHUNK // 16, jbody, (rep_c, prop_c))

        rep_acc, prop_acc = lax.fori_loop(
            0, _P // _NW // _CHUNK, sa_chunk, (rep_acc, prop_acc))

        def dis_chunk(c, caus_c):
            base = wid * (_P // _NW) + c * _CHUNK
            load_pair_cols(dis_hbm, base)
            h1 = pltpu.async_copy(st.at[ia], ga, sem)
            h2 = pltpu.async_copy(st.at[ib], gb, sem)
            h1.wait(); h2.wait()

            def jbody(j, caus_j):
                return caus_j + jnp.exp(-sqdist_16(ga, gb, j))

            return lax.fori_loop(0, _CHUNK // 16, jbody, caus_c)

        caus_acc = lax.fori_loop(0, _P // _NW // _CHUNK, dis_chunk, caus_acc)

        def ref_chunk(c, ref_c):
            base = wid * (_PR // _NW) + c * _CHUNK
            load_pair_cols(ref_hbm, base)
            h1 = pltpu.async_copy(st.at[ia], ga, sem)
            h2 = pltpu.async_copy(st.at[ib], gb, sem)
            h1.wait(); h2.wait()

            def jbody(j, ref_j):
                return ref_j + sqdist_16(ga, gb, j)

            return lax.fori_loop(0, _CHUNK // 16, jbody, ref_c)

        ref_acc = lax.fori_loop(0, _PR // _NW // _CHUNK, ref_chunk, ref_acc)

    outb[0] = caus_acc
    outb[1] = prop_acc
    outb[2] = rep_acc
    outb[3] = ref_acc
    for k in range(4, 8):
        outb[k] = zf
    pltpu.sync_copy(outb, out_hbm.at[wid])


@functools.partial(
    pl.kernel,
    out_type=jax.ShapeDtypeStruct((_NW, 8, 16), jnp.float32),
    mesh=plsc.VectorSubcoreMesh(core_axis_name="c", subcore_axis_name="s"),
    scratch_types=[
        pltpu.VMEM((2 * _CHUNK,), jnp.int32),
        pltpu.VMEM((_CHUNK,), jnp.int32),
        pltpu.VMEM((_CHUNK,), jnp.int32),
        pltpu.VMEM((_CHUNK, _D), jnp.float32),
        pltpu.VMEM((_CHUNK, _D), jnp.float32),
        pltpu.VMEM((_CHUNK, _D), jnp.float32),
        pltpu.VMEM((_CHUNK, _D), jnp.float32),
        pltpu.VMEM((8, 16), jnp.float32),
        pltpu.SemaphoreType.DMA,
    ],
    compiler_params=pltpu.CompilerParams(
        use_tc_tiling_on_sc=False,
        needs_layout_passes=False,
    ),
)
def _sc_pairs(s_hbm, ns_hbm, ps_hbm, nps_hbm, dis_hbm, sa_hbm, ref_hbm,
              out_hbm, prs, ia, ib, ga, gb, gc, gd, outb, sem):
    _sc_pair_kernel(s_hbm, ns_hbm, ps_hbm, nps_hbm, dis_hbm, sa_hbm, ref_hbm,
                    out_hbm, prs, ia, ib, ga, gb, gc, gd, outb, sem)


def _tc_dense_kernel(s_ref, p_ref, n_ref, ns_ref, nps_ref, w_ref, o_ref):
    s = s_ref[...]
    d1 = ns_ref[...] - s
    t1 = jnp.sum(d1 * d1)
    p = p_ref[...]
    d2 = nps_ref[...] - p
    t2 = jnp.sum(d2 * d2)
    dp = jnp.sum((s - p) ** 2, axis=1)
    dn = jnp.sum((s - n_ref[...]) ** 2, axis=1)
    tri = jnp.sum(jnp.maximum(dp - dn + _ALPHA, 0.0))
    l1 = jnp.sum(jnp.abs(w_ref[...]))
    col = lax.broadcasted_iota(jnp.int32, (1, 1, 128), 2)
    row = jnp.where(col == 0, t1 + t2,
                    jnp.where(col == 1, tri,
                              jnp.where(col == 2, l1, 0.0)))
    o_ref[...] = row


_TC_GRID = 32
_RB = _B // _TC_GRID


def _tc_dense(states, p_states, n_states, next_states, next_p_st, w):
    row_spec = pl.BlockSpec((_RB, _D), lambda i: (i, 0))
    return pl.pallas_call(
        _tc_dense_kernel,
        grid=(_TC_GRID,),
        in_specs=[row_spec, row_spec, row_spec, row_spec, row_spec,
                  pl.BlockSpec((256 // _TC_GRID, _D), lambda i: (i, 0))],
        out_specs=pl.BlockSpec((1, 1, 128), lambda i: (i, 0, 0)),
        out_shape=jax.ShapeDtypeStruct((_TC_GRID, 1, 128), jnp.float32),
    )(states, p_states, n_states, next_states, next_p_st, w)


def kernel(states, p_states, n_states, next_states, next_p_st,
           dissimilar_pairs, same_actions_pairs, ref_point_pairs,
           similar_pairs, W):
    del similar_pairs
    dis = dissimilar_pairs.astype(jnp.int32).reshape(-1)
    sap = same_actions_pairs.astype(jnp.int32).reshape(-1)
    rpp = ref_point_pairs.astype(jnp.int32).reshape(-1)

    dense = _tc_dense(states, p_states, n_states, next_states, next_p_st, W)
    partials = _sc_pairs(states, next_states, p_states, next_p_st,
                         dis, sap, rpp)

    temp_sum = jnp.sum(dense[:, 0, 0])
    tri_sum = jnp.sum(dense[:, 0, 1])
    l1_sum = jnp.sum(dense[:, 0, 2])
    pair_sums = jnp.sum(partials, axis=(0, 2))

    total = (
        (_L1_REG / W.size) * l1_sum
        + temp_sum / _B
        + pair_sums[0] / _P
        + pair_sums[1] / _P
        + pair_sums[2] / _P
        + pair_sums[3] / _PR
        + tri_sum / _B
    )
    return total
